# baseline (speedup 1.0000x reference)
import functools

import jax
import jax.numpy as jnp
from jax import lax
from jax.experimental import pallas as pl
from jax.experimental.pallas import tpu as pltpu
from jax.experimental.pallas import tpu_sc as plsc

BATCH = 16384
EMBED_DIM = 64
BLK_W = 128
ROWS_PER_BLK = BLK_W // EMBED_DIM

_info = plsc.get_sparse_core_info()
_NC, _NS, _L = _info.num_cores, _info.num_subcores, _info.num_lanes
_NW = _NC * _NS
_BPW = BATCH // _NW
_PASS = 2
_RPP = _BPW // _PASS
_CPP = _RPP // _L


def _sc_body(uidx_hbm, midx_hbm, utab_hbm, mtab_hbm, out_hbm,
             uidx_v, midx_v, ublk_v, mblk_v, ubi_v, mbi_v, out_v,
             sem_u, sem_m):
    wid = lax.axis_index("s") * _NC + lax.axis_index("c")
    base = wid * _BPW

    pltpu.sync_copy(uidx_hbm.at[pl.ds(base, _BPW)], uidx_v)
    pltpu.sync_copy(midx_hbm.at[pl.ds(base, _BPW)], midx_v)

    def do_pass(p, carry):
        def bi_body(i, c2):
            s = i * _L
            u = uidx_v[pl.ds(p * _RPP + s, _L)]
            m = midx_v[pl.ds(p * _RPP + s, _L)]
            ubi_v[pl.ds(s, _L)] = lax.shift_right_logical(u, 1)
            mbi_v[pl.ds(s, _L)] = lax.shift_right_logical(m, 1)
            return c2

        lax.fori_loop(0, _CPP, bi_body, 0)

        cu = pltpu.async_copy(utab_hbm.at[ubi_v], ublk_v, sem_u)
        cm = pltpu.async_copy(mtab_hbm.at[mbi_v], mblk_v, sem_m)
        cu.wait()
        cm.wait()

        def chunk_body(c, c2):
            row_idx = c * _L + lax.iota(jnp.int32, _L)
            uo = uidx_v[pl.ds(p * _RPP + c * _L, _L)]
            mo = midx_v[pl.ds(p * _RPP + c * _L, _L)]
            ucol0 = (uo & 1) * EMBED_DIM
            mcol0 = (mo & 1) * EMBED_DIM
            acc = jnp.zeros((_L,), jnp.float32)
            for d in range(EMBED_DIM):
                u = plsc.load_gather(ublk_v, [row_idx, ucol0 + d])
                m = plsc.load_gather(mblk_v, [row_idx, mcol0 + d])
                acc = acc + u * m
            out_v[pl.ds(p * _RPP + c * _L, _L)] = acc
            return c2

        lax.fori_loop(0, _CPP, chunk_body, 0)
        return carry

    lax.fori_loop(0, _PASS, do_pass, 0)
    pltpu.sync_copy(out_v, out_hbm.at[pl.ds(base, _BPW)])


def kernel(user_indices, movie_indices, user_table, movie_table):
    uidx = user_indices.astype(jnp.int32)
    midx = movie_indices.astype(jnp.int32)
    utab = user_table.reshape(-1, BLK_W)
    mtab = movie_table.reshape(-1, BLK_W)
    mesh = plsc.VectorSubcoreMesh(core_axis_name="c", subcore_axis_name="s")
    run = functools.partial(
        pl.kernel,
        mesh=mesh,
        out_type=jax.ShapeDtypeStruct((BATCH,), jnp.float32),
        scratch_types=[
            pltpu.VMEM((_BPW,), jnp.int32),
            pltpu.VMEM((_BPW,), jnp.int32),
            pltpu.VMEM((_RPP, BLK_W), jnp.float32),
            pltpu.VMEM((_RPP, BLK_W), jnp.float32),
            pltpu.VMEM((_RPP,), jnp.int32),
            pltpu.VMEM((_RPP,), jnp.int32),
            pltpu.VMEM((_BPW,), jnp.float32),
            pltpu.SemaphoreType.DMA,
            pltpu.SemaphoreType.DMA,
        ],
        compiler_params=pltpu.CompilerParams(needs_layout_passes=False),
    )(_sc_body)
    return run(uidx, midx, utab, mtab)
